# baseline (speedup 1.0000x reference)
import jax
import jax.numpy as jnp
from jax.experimental import pallas as pl
from jax.experimental.pallas import tpu as pltpu

_INPUT_SIZE = 768
_OUTPUT_SIZE = 768
_NUM_EXPERTS = 8
_TOP_K = 2
_BLK = 512


def _smax(a):
    a = jnp.maximum(a[0:4, :], a[4:8, :])
    a = jnp.maximum(a[0:2, :], a[2:4, :])
    return jnp.maximum(a[0:1, :], a[1:2, :])


def _ssum(a):
    a = a[0:4, :] + a[4:8, :]
    a = a[0:2, :] + a[2:4, :]
    return a[0:1, :] + a[1:2, :]


def _moe_block_kernel(x_ref, gw_ref, w_ref, b_ref, gb_ref, eb_ref,
                      out_ref, idx_ref):
    xb = x_ref[...]
    pp = jnp.concatenate(
        [gw_ref[...], w_ref[:, 0, :], w_ref[:, 1, :]], axis=0)
    r_t = jax.lax.dot_general(
        pp, xb,
        dimension_numbers=(((1,), (1,)), ((), ())),
        preferred_element_type=jnp.float32,
    )
    gb2 = gb_ref[...] + eb_ref[...]
    gb2_col = jnp.transpose(gb2)
    logits = r_t[0:8, :] + gb2_col
    v0 = r_t[8:16, :] + b_ref[:, 0:1]
    v1 = r_t[16:24, :] + b_ref[:, 1:2]

    probs = jax.nn.sigmoid(logits)
    e_iota = jax.lax.broadcasted_iota(jnp.int32, probs.shape, 0)

    key = (jax.lax.bitcast_convert_type(probs, jnp.int32) & jnp.int32(-8)) | (
        jnp.int32(_NUM_EXPERTS - 1) - e_iota)
    k1 = _smax(key)
    i1 = jnp.int32(_NUM_EXPERTS - 1) - (k1 & jnp.int32(_NUM_EXPERTS - 1))
    key2 = jnp.where(key == k1, jnp.int32(0), key)
    k2 = _smax(key2)
    i2 = jnp.int32(_NUM_EXPERTS - 1) - (k2 & jnp.int32(_NUM_EXPERTS - 1))

    oh1 = (e_iota == i1).astype(jnp.float32)
    oh2 = (e_iota == i2).astype(jnp.float32)
    m1 = _ssum(oh1 * probs)
    m2 = _ssum(oh2 * probs)
    val1 = _ssum(oh1 * v0)
    val2 = _ssum(oh2 * v1)
    scal = (m1 * val1 + m2 * val2) / (m1 + m2)

    scal_col = jnp.transpose(scal)
    out_ref[...] = jnp.broadcast_to(scal_col, out_ref.shape)
    ipk = jnp.concatenate([i1, i2], axis=0)
    idx_ref[...] = jnp.transpose(ipk)


def kernel(x, W, b, gate_W, gate_b, expert_biases):
    Bn, Sn, _ = x.shape
    n_tok = Bn * Sn
    xf = x.reshape(n_tok, _INPUT_SIZE)

    gb_2d = gate_b.reshape(1, _NUM_EXPERTS)
    eb_2d = expert_biases.reshape(1, _NUM_EXPERTS)
    grid = (n_tok // _BLK,)
    out, idxp = pl.pallas_call(
        _moe_block_kernel,
        grid=grid,
        in_specs=[
            pl.BlockSpec((_BLK, _INPUT_SIZE), lambda i: (i, 0)),
            pl.BlockSpec((_NUM_EXPERTS, _INPUT_SIZE), lambda i: (0, 0)),
            pl.BlockSpec((_NUM_EXPERTS, 8, _INPUT_SIZE), lambda i: (0, 0, 0)),
            pl.BlockSpec((_NUM_EXPERTS, _OUTPUT_SIZE), lambda i: (0, 0)),
            pl.BlockSpec((1, _NUM_EXPERTS), lambda i: (0, 0)),
            pl.BlockSpec((1, _NUM_EXPERTS), lambda i: (0, 0)),
        ],
        out_specs=[
            pl.BlockSpec((_BLK, _OUTPUT_SIZE), lambda i: (i, 0)),
            pl.BlockSpec((_BLK, _TOP_K), lambda i: (i, 0)),
        ],
        out_shape=[
            jax.ShapeDtypeStruct((n_tok, _OUTPUT_SIZE), jnp.float32),
            jax.ShapeDtypeStruct((n_tok, _TOP_K), jnp.int32),
        ],
        compiler_params=pltpu.CompilerParams(
            dimension_semantics=("parallel",),
        ),
    )(xf, gate_W, W, b, gb_2d, eb_2d)

    final_output = out.reshape(Bn, Sn, _OUTPUT_SIZE)
    top_k_indices = idxp.reshape(Bn, Sn, _TOP_K)
    return (final_output, top_k_indices)


# speedup vs baseline: 1.2927x; 1.2927x over previous
import jax
import jax.numpy as jnp
from jax.experimental import pallas as pl
from jax.experimental.pallas import tpu as pltpu

_INPUT_SIZE = 768
_OUTPUT_SIZE = 768
_NUM_EXPERTS = 8
_TOP_K = 2
_BLK = 2048


def _smax(a):
    a = jnp.maximum(a[0:4, :], a[4:8, :])
    a = jnp.maximum(a[0:2, :], a[2:4, :])
    return jnp.maximum(a[0:1, :], a[1:2, :])


def _ssum(a):
    a = a[0:4, :] + a[4:8, :]
    a = a[0:2, :] + a[2:4, :]
    return a[0:1, :] + a[1:2, :]


def _moe_block_kernel(x_ref, gw_ref, w_ref, b_ref, gb_ref, eb_ref,
                      out_ref, idx_ref):
    xb = x_ref[...]
    pp = jnp.concatenate(
        [gw_ref[...], w_ref[:, 0, :], w_ref[:, 1, :]], axis=0)
    r_t = jax.lax.dot_general(
        pp, xb,
        dimension_numbers=(((1,), (1,)), ((), ())),
        preferred_element_type=jnp.float32,
    )
    gb2 = gb_ref[...] + eb_ref[...]
    gb2_col = jnp.transpose(gb2)
    logits = r_t[0:8, :] + gb2_col
    v0 = r_t[8:16, :] + b_ref[:, 0:1]
    v1 = r_t[16:24, :] + b_ref[:, 1:2]

    probs = jax.nn.sigmoid(logits)
    e_iota = jax.lax.broadcasted_iota(jnp.int32, probs.shape, 0)

    key = (jax.lax.bitcast_convert_type(probs, jnp.int32) & jnp.int32(-8)) | (
        jnp.int32(_NUM_EXPERTS - 1) - e_iota)
    k1 = _smax(key)
    i1 = jnp.int32(_NUM_EXPERTS - 1) - (k1 & jnp.int32(_NUM_EXPERTS - 1))
    key2 = jnp.where(key == k1, jnp.int32(0), key)
    k2 = _smax(key2)
    i2 = jnp.int32(_NUM_EXPERTS - 1) - (k2 & jnp.int32(_NUM_EXPERTS - 1))

    oh1 = (e_iota == i1).astype(jnp.float32)
    oh2 = (e_iota == i2).astype(jnp.float32)
    m1 = _ssum(oh1 * probs)
    m2 = _ssum(oh2 * probs)
    val1 = _ssum(oh1 * v0)
    val2 = _ssum(oh2 * v1)
    scal = (m1 * val1 + m2 * val2) / (m1 + m2)

    scal_col = jnp.transpose(scal)
    out_ref[...] = jnp.broadcast_to(scal_col, out_ref.shape)
    ipk = jnp.concatenate([i1, i2], axis=0)
    idx_ref[...] = jnp.transpose(ipk)


def kernel(x, W, b, gate_W, gate_b, expert_biases):
    Bn, Sn, _ = x.shape
    n_tok = Bn * Sn
    xf = x.reshape(n_tok, _INPUT_SIZE)

    gb_2d = gate_b.reshape(1, _NUM_EXPERTS)
    eb_2d = expert_biases.reshape(1, _NUM_EXPERTS)
    grid = (n_tok // _BLK,)
    out, idxp = pl.pallas_call(
        _moe_block_kernel,
        grid=grid,
        in_specs=[
            pl.BlockSpec((_BLK, _INPUT_SIZE), lambda i: (i, 0)),
            pl.BlockSpec((_NUM_EXPERTS, _INPUT_SIZE), lambda i: (0, 0)),
            pl.BlockSpec((_NUM_EXPERTS, 8, _INPUT_SIZE), lambda i: (0, 0, 0)),
            pl.BlockSpec((_NUM_EXPERTS, _OUTPUT_SIZE), lambda i: (0, 0)),
            pl.BlockSpec((1, _NUM_EXPERTS), lambda i: (0, 0)),
            pl.BlockSpec((1, _NUM_EXPERTS), lambda i: (0, 0)),
        ],
        out_specs=[
            pl.BlockSpec((_BLK, _OUTPUT_SIZE), lambda i: (i, 0)),
            pl.BlockSpec((_BLK, _TOP_K), lambda i: (i, 0)),
        ],
        out_shape=[
            jax.ShapeDtypeStruct((n_tok, _OUTPUT_SIZE), jnp.float32),
            jax.ShapeDtypeStruct((n_tok, _TOP_K), jnp.int32),
        ],
        compiler_params=pltpu.CompilerParams(
            dimension_semantics=("parallel",),
        ),
    )(xf, gate_W, W, b, gb_2d, eb_2d)

    final_output = out.reshape(Bn, Sn, _OUTPUT_SIZE)
    top_k_indices = idxp.reshape(Bn, Sn, _TOP_K)
    return (final_output, top_k_indices)
